# baseline (speedup 1.0000x reference)
import jax
import jax.numpy as jnp
from jax import lax
from jax.experimental import pallas as pl
from jax.experimental.pallas import tpu as pltpu

CH = 4000
K = 4


def _chunk_copy(hbm_ref, bufs_ref, sems_ref, c):
    slot = lax.rem(c, K)
    return pltpu.make_async_copy(
        hbm_ref.at[pl.ds(c * CH, CH), :],
        bufs_ref.at[slot],
        sems_ref.at[slot],
    )


def _mlp_kernel(rx_hbm, tx_hbm,
                W_enc_ref, b_enc_ref,
                Wy1_ref, by1_ref, Wy2_ref, by2_ref,
                Wac1_ref, bac1_ref, Wac2_ref, bac2_ref,
                outy_ref, outac_ref,
                rbufs, tbufs, rsems, tsems,
                MyT_ref, cy_ref, MacT_ref, cac_ref):
    i = pl.program_id(0)
    nc = pl.num_programs(0)

    @pl.when(i == 0)
    def _prologue():
        MyT_ref[...] = lax.dot_general(
            W_enc_ref[...], Wy1_ref[...], (((0,), (1,)), ((), ())),
            preferred_element_type=jnp.float32)
        cy_ref[...] = lax.dot_general(
            b_enc_ref[...], Wy1_ref[...], (((1,), (1,)), ((), ())),
            preferred_element_type=jnp.float32) + by1_ref[...]
        MacT_ref[...] = lax.dot_general(
            W_enc_ref[...], Wac1_ref[...], (((0,), (1,)), ((), ())),
            preferred_element_type=jnp.float32)
        cac_ref[...] = lax.dot_general(
            b_enc_ref[...], Wac1_ref[...], (((1,), (1,)), ((), ())),
            preferred_element_type=jnp.float32) + bac1_ref[...]
        for k in range(K):
            _chunk_copy(rx_hbm, rbufs, rsems, k).start()
            _chunk_copy(tx_hbm, tbufs, tsems, k).start()

    @pl.when(jnp.logical_and(i > 0, i + K - 1 < nc))
    def _refill():
        _chunk_copy(rx_hbm, rbufs, rsems, i + K - 1).start()
        _chunk_copy(tx_hbm, tbufs, tsems, i + K - 1).start()

    _chunk_copy(rx_hbm, rbufs, rsems, i).wait()
    _chunk_copy(tx_hbm, tbufs, tsems, i).wait()
    slot = lax.rem(i, K)

    hy = jnp.maximum(
        jnp.dot(rbufs[slot], MyT_ref[...],
                preferred_element_type=jnp.float32) + cy_ref[...], 0.0)
    outy_ref[0] = lax.dot_general(
        Wy2_ref[...], hy, (((1,), (1,)), ((), ())),
        preferred_element_type=jnp.float32) + by2_ref[...]

    hac = jnp.maximum(
        jnp.dot(tbufs[slot], MacT_ref[...],
                preferred_element_type=jnp.float32) + cac_ref[...], 0.0)
    outac_ref[0] = lax.dot_general(
        Wac2_ref[...], hac, (((1,), (1,)), ((), ())),
        preferred_element_type=jnp.float32) + bac2_ref[...]


def kernel(mol_x, reaction_x, target_x, W_enc, b_enc, Wa1, ba1, Wa2, ba2,
           W_upd, b_upd, Wy1, by1, Wy2, by2, Wac1, bac1, Wac2, bac2):
    del mol_x, Wa1, ba1, Wa2, ba2, W_upd, b_upd
    n = reaction_x.shape[0]
    nc = n // CH

    b_enc2 = b_enc.reshape(1, -1)
    by1_2 = by1.reshape(1, -1)
    by2_2 = by2.reshape(1, 1)
    bac1_2 = bac1.reshape(1, -1)
    bac2_2 = bac2.reshape(1, 1)

    hbm_spec = pl.BlockSpec(memory_space=pltpu.MemorySpace.HBM)
    out_spec = pl.BlockSpec((1, 1, CH), lambda i: (i, 0, 0))

    def whole(shape):
        return pl.BlockSpec(shape, lambda i: tuple(0 for _ in shape))

    outy, outac = pl.pallas_call(
        _mlp_kernel,
        grid=(nc,),
        in_specs=[
            hbm_spec, hbm_spec,
            whole((128, 128)), whole((1, 128)),
            whole((64, 128)), whole((1, 64)), whole((1, 64)), whole((1, 1)),
            whole((64, 128)), whole((1, 64)), whole((1, 64)), whole((1, 1)),
        ],
        out_specs=[out_spec, out_spec],
        out_shape=[
            jax.ShapeDtypeStruct((nc, 1, CH), jnp.float32),
            jax.ShapeDtypeStruct((nc, 1, CH), jnp.float32),
        ],
        scratch_shapes=[
            pltpu.VMEM((K, CH, 128), jnp.float32),
            pltpu.VMEM((K, CH, 128), jnp.float32),
            pltpu.SemaphoreType.DMA((K,)),
            pltpu.SemaphoreType.DMA((K,)),
            pltpu.VMEM((128, 64), jnp.float32),
            pltpu.VMEM((1, 64), jnp.float32),
            pltpu.VMEM((128, 64), jnp.float32),
            pltpu.VMEM((1, 64), jnp.float32),
        ],
        compiler_params=pltpu.CompilerParams(
            dimension_semantics=("arbitrary",)),
    )(reaction_x, target_x,
      W_enc, b_enc2,
      Wy1, by1_2, Wy2, by2_2,
      Wac1, bac1_2, Wac2, bac2_2)

    return (outy.reshape(-1), outac.reshape(-1))
